# baseline (speedup 1.0000x reference)
import functools

import jax
import jax.numpy as jnp
import numpy as np
from jax import lax
from jax.experimental import pallas as pl
from jax.experimental.pallas import tpu as pltpu
from jax.experimental.pallas import tpu_sc as plsc

ROWS = 16384
HIST = 50
HIST_PAD = 128
D = 32
LANES = 16
RGRP = 8

_info = plsc.get_sparse_core_info()
_NC, _NS = _info.num_cores, _info.num_subcores
NW = _NC * _NS
ROWS_W = ROWS // NW
CHUNK = 32
NCH = ROWS_W // CHUNK
NGRP = CHUNK // RGRP
PATN = RGRP * HIST
NVREG = PATN // LANES

_p = np.arange(PATN, dtype=np.int32)
_ROW_PAT = _p // HIST
_COL_PAT = _p % HIST

_mesh = plsc.VectorSubcoreMesh(core_axis_name="c", subcore_axis_name="s")


@functools.partial(
    pl.kernel,
    mesh=_mesh,
    out_type=jax.ShapeDtypeStruct((ROWS, HIST, D), jnp.float32),
    scratch_types=[
        pltpu.VMEM((PATN,), jnp.int32),
        pltpu.VMEM((PATN,), jnp.int32),
        pltpu.VMEM((CHUNK, HIST_PAD), jnp.int32),
        pltpu.VMEM((CHUNK, HIST_PAD), jnp.int32),
        pltpu.VMEM((CHUNK, HIST), jnp.int32),
        pltpu.VMEM((CHUNK, HIST), jnp.int32),
        pltpu.VMEM((CHUNK, HIST, D), jnp.float32),
        pltpu.VMEM((CHUNK, HIST, D), jnp.float32),
        pltpu.SemaphoreType.DMA,
        pltpu.SemaphoreType.DMA,
        pltpu.SemaphoreType.DMA,
        pltpu.SemaphoreType.DMA,
        pltpu.SemaphoreType.DMA,
        pltpu.SemaphoreType.DMA,
    ],
    compiler_params=pltpu.CompilerParams(
        use_tc_tiling_on_sc=False, needs_layout_passes=False
    ),
)
def _gather_kernel(idx_hbm, table_hbm, rpat_hbm, cpat_hbm, out_hbm,
                   rpat_v, cpat_v, pad0, pad1, den0, den1, buf0, buf1,
                   i0, i1, g0, g1, s0, s1):
    wid = lax.axis_index("s") * _NC + lax.axis_index("c")
    base = wid * ROWS_W
    pads, dens, isems = (pad0, pad1), (den0, den1), (i0, i1)
    bufs, gsems, ssems = (buf0, buf1), (g0, g1), (s0, s1)
    pltpu.sync_copy(rpat_hbm, rpat_v)
    pltpu.sync_copy(cpat_hbm, cpat_v)

    def start_idx(g):
        b = g % 2
        return pltpu.async_copy(
            idx_hbm.at[pl.ds(base + g * CHUNK, CHUNK)], pads[b], isems[b]
        )

    def compact(b):
        def body(grp, carry):
            src = pads[b].at[pl.ds(grp * RGRP, RGRP)]
            dst = dens[b].at[pl.ds(grp * RGRP, RGRP)]
            for k in range(NVREG):
                r = rpat_v[pl.ds(k * LANES, LANES)]
                c = cpat_v[pl.ds(k * LANES, LANES)]
                vals = plsc.load_gather(src, [r, c])
                plsc.store_scatter(dst, [r, c], vals)
            return carry
        lax.fori_loop(0, NGRP, body, 0)

    def do_chunk(g, icp):
        b = g % 2
        icp.wait()
        compact(b)
        for r in range(CHUNK):
            pltpu.async_copy(table_hbm.at[dens[b].at[r]], bufs[b].at[r], gsems[b])
        nicp = start_idx(g + 1) if g + 1 < NCH else None
        pltpu.make_async_copy(
            out_hbm.at[pl.ds(base + g * CHUNK, CHUNK)], bufs[b], gsems[b]
        ).wait()
        st = pltpu.async_copy(
            bufs[b], out_hbm.at[pl.ds(base + g * CHUNK, CHUNK)], ssems[b]
        )
        return st, nicp

    icp = start_idx(0)
    stores = [None, None]
    for g in range(NCH):
        b = g % 2
        if stores[b] is not None:
            stores[b].wait()
        stores[b], icp = do_chunk(g, icp)
    stores[0].wait()
    stores[1].wait()


def kernel(inputs, table):
    padded = jnp.pad(inputs, ((0, 0), (0, HIST_PAD - HIST)))
    rpat = jnp.asarray(_ROW_PAT)
    cpat = jnp.asarray(_COL_PAT)
    return _gather_kernel(padded, table, rpat, cpat)


# speedup vs baseline: 1.0047x; 1.0047x over previous
import functools

import jax
import jax.numpy as jnp
from jax import lax
from jax.experimental import pallas as pl
from jax.experimental.pallas import tpu as pltpu
from jax.experimental.pallas import tpu_sc as plsc

ROWS = 16384
HIST = 50
D = 32
LANES = 16
VOCAB1 = 1001

_info = plsc.get_sparse_core_info()
_NC, _NS = _info.num_cores, _info.num_subcores
NW = _NC * _NS
NBT = ROWS // 128
BT_W = NBT // NW
NBUF = 2
HPAIRS = HIST // NBUF

_mesh = plsc.VectorSubcoreMesh(core_axis_name="c", subcore_axis_name="s")


@functools.partial(
    pl.kernel,
    mesh=_mesh,
    out_type=jax.ShapeDtypeStruct((HIST, D // 8, NBT, 8, 128), jnp.float32),
    scratch_types=[
        pltpu.VMEM((VOCAB1, D), jnp.float32),
        pltpu.VMEM((BT_W, HIST, 128), jnp.int32),
        pltpu.VMEM((D // 8, 8, 128), jnp.float32),
        pltpu.VMEM((D // 8, 8, 128), jnp.float32),
        pltpu.SemaphoreType.DMA,
        pltpu.SemaphoreType.DMA,
        pltpu.SemaphoreType.DMA,
    ],
    compiler_params=pltpu.CompilerParams(
        use_tc_tiling_on_sc=False, needs_layout_passes=False
    ),
)
def _gather_kernel(idx_hbm, table_hbm, out_hbm,
                   tab_v, idx_v, buf0, buf1, isem, s0, s1):
    wid = lax.axis_index("s") * _NC + lax.axis_index("c")
    bufs, ssems = (buf0, buf1), (s0, s1)
    pltpu.sync_copy(table_hbm, tab_v)
    icps = [
        pltpu.async_copy(
            idx_hbm.at[:, pl.ds((wid * BT_W + j) * 128, 128)],
            idx_v.at[j], isem,
        )
        for j in range(BT_W)
    ]
    for c in icps:
        c.wait()
    zero16 = lax.iota(jnp.int32, LANES) * 0

    def compute_tile(j, h, idxb):
        buf = bufs[j]
        ivs = [idxb[h, pl.ds(k * LANES, LANES)] for k in range(128 // LANES)]
        for d in range(D):
            dvec = zero16 + d
            dst = buf.at[d // 8].at[d % 8]
            for k in range(128 // LANES):
                dst[pl.ds(k * LANES, LANES)] = plsc.load_gather(
                    tab_v, [ivs[k], dvec]
                )

    def drain(j, bt):
        pltpu.make_async_copy(
            out_hbm.at[0, slice(None), bt], bufs[j], ssems[j]
        ).wait()

    def body(o, carry):
        jb = o // HPAIRS
        oh = o - jb * HPAIRS
        bt = wid * BT_W + jb
        idxb = idx_v.at[jb]

        @pl.when(o > 0)
        def _():
            for j in range(NBUF):
                drain(j, bt)

        for j in range(NBUF):
            h = oh * NBUF + j
            compute_tile(j, h, idxb)
            pltpu.async_copy(bufs[j], out_hbm.at[h, slice(None), bt], ssems[j])
        return carry

    lax.fori_loop(0, BT_W * HPAIRS, body, 0)
    for j in range(NBUF):
        drain(j, 0)


def kernel(inputs, table):
    idx_t = inputs.T
    out5 = _gather_kernel(idx_t, table)
    return out5.transpose(2, 4, 0, 1, 3).reshape(ROWS, HIST, D)


# speedup vs baseline: 1.2426x; 1.2368x over previous
import functools

import jax
import jax.numpy as jnp
from jax import lax
from jax.experimental import pallas as pl
from jax.experimental.pallas import tpu as pltpu
from jax.experimental.pallas import tpu_sc as plsc

ROWS = 16384
HIST = 50
D = 32
LANES = 16
VOCAB1 = 1001

_info = plsc.get_sparse_core_info()
_NC, _NS = _info.num_cores, _info.num_subcores
NW = _NC * _NS
NBT = ROWS // 128
BT_W = NBT // NW
NBUF = 2
HPAIRS = HIST // NBUF

_mesh = plsc.VectorSubcoreMesh(core_axis_name="c", subcore_axis_name="s")


@functools.partial(
    pl.kernel,
    mesh=_mesh,
    out_type=jax.ShapeDtypeStruct((HIST, D // 8, NBT, 8, 128), jnp.float32),
    scratch_types=[
        pltpu.VMEM((VOCAB1, D), jnp.float32),
        pltpu.VMEM((BT_W, HIST, 128), jnp.int32),
        pltpu.VMEM((D // 8, 8, 128), jnp.float32),
        pltpu.VMEM((D // 8, 8, 128), jnp.float32),
        pltpu.SemaphoreType.DMA,
        pltpu.SemaphoreType.DMA,
        pltpu.SemaphoreType.DMA,
    ],
    compiler_params=pltpu.CompilerParams(
        use_tc_tiling_on_sc=False, needs_layout_passes=False
    ),
)
def _gather_kernel(idx_hbm, table_hbm, out_hbm,
                   tab_v, idx_v, buf0, buf1, isem, s0, s1):
    wid = lax.axis_index("s") * _NC + lax.axis_index("c")
    bufs, ssems = (buf0, buf1), (s0, s1)
    pltpu.sync_copy(table_hbm, tab_v)
    icps = [
        pltpu.async_copy(
            idx_hbm.at[:, pl.ds((wid * BT_W + j) * 128, 128)],
            idx_v.at[j], isem,
        )
        for j in range(BT_W)
    ]
    for c in icps:
        c.wait()
    zero16 = lax.iota(jnp.int32, LANES) * 0

    def compute_tile(j, h, idxb):
        buf = bufs[j]
        ivs = [idxb[h, pl.ds(k * LANES, LANES)] for k in range(128 // LANES)]
        for d in range(D):
            dvec = zero16 + d
            vals = [
                plsc.load_gather(tab_v, [ivs[k], dvec])
                for k in range(128 // LANES)
            ]
            dst = buf.at[d // 8].at[d % 8]
            for k in range(128 // LANES):
                dst[pl.ds(k * LANES, LANES)] = vals[k]

    def drain(j, bt):
        pltpu.make_async_copy(
            out_hbm.at[0, slice(None), bt], bufs[j], ssems[j]
        ).wait()

    def body(o, carry):
        jb = o // HPAIRS
        oh = o - jb * HPAIRS
        bt = wid * BT_W + jb
        idxb = idx_v.at[jb]

        @pl.when(o > 0)
        def _():
            for j in range(NBUF):
                drain(j, bt)

        for j in range(NBUF):
            h = oh * NBUF + j
            compute_tile(j, h, idxb)
            pltpu.async_copy(bufs[j], out_hbm.at[h, slice(None), bt], ssems[j])
        return carry

    lax.fori_loop(0, BT_W * HPAIRS, body, 0)
    for j in range(NBUF):
        drain(j, 0)


def kernel(inputs, table):
    idx_t = inputs.T
    out5 = _gather_kernel(idx_t, table)
    return out5.transpose(2, 4, 0, 1, 3).reshape(ROWS, HIST, D)


# speedup vs baseline: 5.7882x; 4.6581x over previous
import functools

import jax
import jax.numpy as jnp
from jax import lax
from jax.experimental import pallas as pl
from jax.experimental.pallas import tpu as pltpu
from jax.experimental.pallas import tpu_sc as plsc

ROWS = 16384
HIST = 50
D = 32
LANES = 16
VOCAB1 = 1001

_info = plsc.get_sparse_core_info()
_NC, _NS = _info.num_cores, _info.num_subcores
NW = _NC * _NS
NBT = ROWS // 128
BT_W = NBT // NW
NBUF = 2
HPAIRS = HIST // NBUF

_mesh = plsc.VectorSubcoreMesh(core_axis_name="c", subcore_axis_name="s")


@functools.partial(
    pl.kernel,
    mesh=_mesh,
    out_type=jax.ShapeDtypeStruct((HIST, D // 8, NBT, 8, 128), jnp.float32),
    scratch_types=[
        pltpu.VMEM((D, VOCAB1), jnp.float32),
        pltpu.VMEM((BT_W, HIST, 128), jnp.int32),
        pltpu.VMEM((D // 8, 8, 128), jnp.float32),
        pltpu.VMEM((D // 8, 8, 128), jnp.float32),
        pltpu.SemaphoreType.DMA,
        pltpu.SemaphoreType.DMA,
        pltpu.SemaphoreType.DMA,
    ],
    compiler_params=pltpu.CompilerParams(
        use_tc_tiling_on_sc=False, needs_layout_passes=False
    ),
)
def _gather_kernel(idx_hbm, table_hbm, out_hbm,
                   tab_v, idx_v, buf0, buf1, isem, s0, s1):
    wid = lax.axis_index("s") * _NC + lax.axis_index("c")
    bufs, ssems = (buf0, buf1), (s0, s1)
    pltpu.sync_copy(table_hbm, tab_v)
    icps = [
        pltpu.async_copy(
            idx_hbm.at[:, pl.ds((wid * BT_W + j) * 128, 128)],
            idx_v.at[j], isem,
        )
        for j in range(BT_W)
    ]
    for c in icps:
        c.wait()
    zero16 = lax.iota(jnp.int32, LANES) * 0

    def compute_tile(j, h, idxb):
        buf = bufs[j]
        ivs = [idxb[h, pl.ds(k * LANES, LANES)] for k in range(128 // LANES)]
        for d in range(D):
            dvec = zero16 + d
            vals = [
                plsc.load_gather(tab_v, [dvec, ivs[k]])
                for k in range(128 // LANES)
            ]
            dst = buf.at[d // 8].at[d % 8]
            for k in range(128 // LANES):
                dst[pl.ds(k * LANES, LANES)] = vals[k]

    def drain(j, bt):
        pltpu.make_async_copy(
            out_hbm.at[0, slice(None), bt], bufs[j], ssems[j]
        ).wait()

    def body(o, carry):
        jb = o // HPAIRS
        oh = o - jb * HPAIRS
        bt = wid * BT_W + jb
        idxb = idx_v.at[jb]

        @pl.when(o > 0)
        def _():
            for j in range(NBUF):
                drain(j, bt)

        for j in range(NBUF):
            h = oh * NBUF + j
            compute_tile(j, h, idxb)
            pltpu.async_copy(bufs[j], out_hbm.at[h, slice(None), bt], ssems[j])
        return carry

    lax.fori_loop(0, BT_W * HPAIRS, body, 0)
    for j in range(NBUF):
        drain(j, 0)


def kernel(inputs, table):
    idx_t = inputs.T
    tab_t = table.T
    out5 = _gather_kernel(idx_t, tab_t)
    return out5.transpose(2, 4, 0, 1, 3).reshape(ROWS, HIST, D)
